# baseline (speedup 1.0000x reference)
import functools

import numpy as np
import jax
import jax.numpy as jnp
from jax import lax
from jax.experimental import pallas as pl
from jax.experimental.pallas import tpu as pltpu
from jax.experimental.pallas import tpu_sc as plsc

T = 2048
C = 1024
H = 16
HD = 64
NE = 8
DFF = 4096
EPAD = 128

BT = 256
NT = T // BT
FT = 256
NFT = T // FT
DC = 1024
NDC = DFF // DC

SC_CORES = 2
SC_SUBCORES = 16
NW = SC_CORES * SC_SUBCORES
RPW = T // NW

_SCALE = float(C) ** -0.5
_NEG = -1e30


def _rotary_const():
    t = np.arange(T, dtype=np.float32)
    f = np.arange(0, C, 2, dtype=np.float32) / C
    ang = 2.0 * np.pi * t[:, None] * f[None, :]
    emb = np.zeros((T, C), np.float32)
    emb[:, 0::2] = np.sin(ang)
    emb[:, 1::2] = np.cos(ang)
    return jnp.asarray(emb)


def _ln(x, g, b):
    m = jnp.mean(x, axis=-1, keepdims=True)
    v = jnp.mean((x - m) ** 2, axis=-1, keepdims=True)
    return (x - m) * lax.rsqrt(v + 1e-5) * g + b


def _dot_nt(a, b):
    return lax.dot_general(a, b, (((1,), (1,)), ((), ())),
                           preferred_element_type=jnp.float32)


def _dot_nn(a, b):
    return lax.dot_general(a, b, (((1,), (0,)), ((), ())),
                           preferred_element_type=jnp.float32)


def _dot_nt_bf(a, b):
    return lax.dot_general(a.astype(jnp.bfloat16), b.astype(jnp.bfloat16),
                           (((1,), (1,)), ((), ())),
                           preferred_element_type=jnp.float32)


def _qkv_body(x_ref, rot_ref, pos_ref, g_ref, b_ref, wq_ref, wk_ref, wv_ref,
              q_ref, k_ref, v_ref):
    h = _ln(x_ref[...] + rot_ref[...] + pos_ref[...], g_ref[...], b_ref[...])
    q_ref[...] = _dot_nt(h, wq_ref[...])
    k_ref[...] = _dot_nt(h, wk_ref[...])
    v_ref[...] = _dot_nt(h, wv_ref[...])


def _qkv(x2d, rot, pos, g, b, wq2, wk2, wv2):
    row = pl.BlockSpec((BT, C), lambda i: (i, 0))
    full = pl.BlockSpec((C, C), lambda i: (0, 0))
    vec = pl.BlockSpec((1, C), lambda i: (0, 0))
    return pl.pallas_call(
        _qkv_body,
        grid=(NT,),
        in_specs=[row, row, row, vec, vec, full, full, full],
        out_specs=[row, row, row],
        out_shape=[jax.ShapeDtypeStruct((T, C), jnp.float32)] * 3,
    )(x2d, rot, pos, g, b, wq2, wk2, wv2)


HP = H // 2


def _attn_body(q_ref, k_ref, v_ref, o_ref):
    tq = pl.program_id(1)
    q = q_ref[...]
    ri = lax.broadcasted_iota(jnp.int32, (BT, BT), 0)
    cj = lax.broadcasted_iota(jnp.int32, (BT, BT), 1)
    noncausal = cj > ri
    outs = []
    for hh in range(2):
        qh = q[:, hh * HD:(hh + 1) * HD]

        def body(j, carry, hh=hh, qh=qh):
            m, l, acc = carry
            kv = k_ref[pl.ds(j * BT, BT), :]
            kh = kv[:, hh * HD:(hh + 1) * HD]
            vv = v_ref[pl.ds(j * BT, BT), :]
            vh = vv[:, hh * HD:(hh + 1) * HD]
            s = _dot_nt(qh, kh) * _SCALE
            s = jnp.where((j == tq) & noncausal, _NEG, s)
            m2 = jnp.maximum(m, jnp.max(s, axis=1, keepdims=True))
            alpha = jnp.exp(m - m2)
            p = jnp.exp(s - m2)
            l2 = l * alpha + jnp.sum(p, axis=1, keepdims=True)
            acc2 = acc * alpha + _dot_nn(p, vh)
            return (m2, l2, acc2)

        init = (jnp.full((BT, 1), _NEG, jnp.float32),
                jnp.zeros((BT, 1), jnp.float32),
                jnp.zeros((BT, HD), jnp.float32))
        m, l, acc = lax.fori_loop(0, tq + 1, body, init)
        outs.append(acc / l)
    o_ref[...] = jnp.concatenate(outs, axis=1)


def _attn(q2d, k2d, v2d):
    qspec = pl.BlockSpec((BT, 2 * HD), lambda hp, tq: (tq, hp))
    kvspec = pl.BlockSpec((T, 2 * HD), lambda hp, tq: (0, hp))
    return pl.pallas_call(
        _attn_body,
        grid=(HP, NT),
        in_specs=[qspec, kvspec, kvspec],
        out_specs=pl.BlockSpec((BT, 2 * HD), lambda hp, tq: (tq, hp)),
        out_shape=jax.ShapeDtypeStruct((T, C), jnp.float32),
    )(q2d, k2d, v2d)


def _proj_body(x_ref, a_ref, wot_ref, bo_ref, g_ref, b_ref, wg_ref,
               x2_ref, gl_ref):
    x2 = x_ref[...] + _dot_nn(a_ref[...], wot_ref[...]) + bo_ref[...]
    x2_ref[...] = x2
    h2 = _ln(x2, g_ref[...], b_ref[...])
    gl_ref[...] = _dot_nt(h2, wg_ref[...])


def _proj(x2d, att, wot, bo, g, b, wgp):
    row = pl.BlockSpec((BT, C), lambda i: (i, 0))
    vec = pl.BlockSpec((1, C), lambda i: (0, 0))
    return pl.pallas_call(
        _proj_body,
        grid=(NT,),
        in_specs=[row, row,
                  pl.BlockSpec((C, C), lambda i: (0, 0)),
                  vec, vec, vec,
                  pl.BlockSpec((EPAD, C), lambda i: (0, 0))],
        out_specs=[row, pl.BlockSpec((BT, EPAD), lambda i: (i, 0))],
        out_shape=[jax.ShapeDtypeStruct((T, C), jnp.float32),
                   jax.ShapeDtypeStruct((T, EPAD), jnp.float32)],
    )(x2d, att, wot, bo, g, b, wgp)


def _route_body(gl_ref, p_ref, se_ref):
    gl = gl_ref[...]
    cols = lax.broadcasted_iota(jnp.int32, (T, EPAD), 1)
    gl = jnp.where(cols < NE, gl, _NEG)
    m = jnp.max(gl, axis=1, keepdims=True)
    sel = jnp.min(jnp.where(gl == m, cols, EPAD), axis=1, keepdims=True)
    oh = (cols == sel).astype(jnp.float32)
    ri = lax.broadcasted_iota(jnp.int32, (T, T), 0)
    ci = lax.broadcasted_iota(jnp.int32, (T, T), 1)
    lmat = (ci < ri).astype(jnp.float32)
    cs = lax.dot_general(lmat, oh, (((1,), (0,)), ((), ())),
                         preferred_element_type=jnp.float32,
                         precision=lax.Precision.HIGHEST)
    rank = jnp.sum(oh * cs, axis=1, keepdims=True)
    counts = jnp.sum(oh, axis=0, keepdims=True)
    ei = lax.broadcasted_iota(jnp.int32, (EPAD, EPAD), 0)
    ej = lax.broadcasted_iota(jnp.int32, (EPAD, EPAD), 1)
    umat = (ei < ej).astype(jnp.float32)
    offs = lax.dot_general(counts, umat, (((1,), (0,)), ((), ())),
                           preferred_element_type=jnp.float32,
                           precision=lax.Precision.HIGHEST)
    pos = jnp.sum(oh * offs, axis=1, keepdims=True) + rank
    p_ref[...] = jnp.broadcast_to(pos.astype(jnp.int32), (T, EPAD))
    ends = offs + counts
    shmat = ((ei + NE) == ej).astype(jnp.float32)
    ends_sh = lax.dot_general(ends, shmat, (((1,), (0,)), ((), ())),
                              preferred_element_type=jnp.float32,
                              precision=lax.Precision.HIGHEST)
    ecol = lax.broadcasted_iota(jnp.int32, (1, EPAD), 1)
    packed = jnp.where(ecol < NE, offs,
                       jnp.where(ecol < 2 * NE, ends_sh, 0.0))
    se_ref[...] = packed.astype(jnp.int32)


def _route(gl):
    return pl.pallas_call(
        _route_body,
        grid=(1,),
        in_specs=[pl.BlockSpec((T, EPAD), lambda i: (0, 0))],
        out_specs=[pl.BlockSpec((T, EPAD), lambda i: (0, 0)),
                   pl.BlockSpec((1, EPAD), lambda i: (0, 0))],
        out_shape=[jax.ShapeDtypeStruct((T, EPAD), jnp.int32),
                   jax.ShapeDtypeStruct((1, EPAD), jnp.int32)],
    )(gl)


@functools.lru_cache(maxsize=None)
def _sc_kernels():
    mesh = plsc.VectorSubcoreMesh(core_axis_name="c", subcore_axis_name="s")
    common = dict(
        out_type=jax.ShapeDtypeStruct((T, C), jnp.float32),
        mesh=mesh,
        scratch_types=[
            pltpu.VMEM((RPW,), jnp.int32),
            pltpu.VMEM((RPW, C), jnp.float32),
            pltpu.SemaphoreType.DMA,
        ],
    )

    @functools.partial(pl.kernel, **common)
    def scatter_rows(x_hbm, p_hbm, out_hbm, idx_v, rows_v, sem):
        wid = lax.axis_index("s") * SC_CORES + lax.axis_index("c")
        base = wid * RPW
        pltpu.sync_copy(p_hbm.at[pl.ds(base, RPW)], idx_v)
        pltpu.sync_copy(x_hbm.at[pl.ds(base, RPW)], rows_v)
        pltpu.async_copy(rows_v, out_hbm.at[idx_v], sem).wait()

    @functools.partial(pl.kernel, **common)
    def gather_rows(ys_hbm, p_hbm, out_hbm, idx_v, rows_v, sem):
        wid = lax.axis_index("s") * SC_CORES + lax.axis_index("c")
        base = wid * RPW
        pltpu.sync_copy(p_hbm.at[pl.ds(base, RPW)], idx_v)
        pltpu.async_copy(ys_hbm.at[idx_v], rows_v, sem).wait()
        pltpu.sync_copy(rows_v, out_hbm.at[pl.ds(base, RPW)])

    return scatter_rows, gather_rows


def _sc_scatter_rows(x2, p):
    return _sc_kernels()[0](x2, p)


def _sc_gather_rows(ys, p):
    return _sc_kernels()[1](ys, p)


def _ffn_body(s_ref, e_ref, xs_ref, w1_ref, b1_ref, w2_ref, b2_ref,
              g_ref, b_ref, out_ref, h2_ref):
    e = pl.program_id(0)
    d = pl.program_id(1)
    ti = pl.program_id(2)

    @pl.when((e == 0) & (d == 0) & (ti == 0))
    def _init():
        xs = xs_ref[...]
        out_ref[...] = xs
        h2_ref[...] = _ln(xs, g_ref[...], b_ref[...])

    seg_s = s_ref[e]
    seg_e = e_ref[e]
    t = jnp.minimum(seg_s // FT + ti, NFT - 1)
    row0 = t * FT
    valid = (seg_s < seg_e) & (seg_s // FT + ti <= (seg_e - 1) // FT)

    @pl.when(valid)
    def _step():
        h2 = h2_ref[pl.ds(row0, FT), :]
        a = jnp.maximum(_dot_nt_bf(h2, w1_ref[0]) + b1_ref[0], 0.0)
        contrib = _dot_nt_bf(a, w2_ref[0])
        contrib = contrib + jnp.where(d == 0, 1.0, 0.0) * b2_ref[0]
        lo = jnp.maximum(seg_s - row0, 0)
        hi = jnp.minimum(seg_e - row0, FT)
        ri = lax.broadcasted_iota(jnp.int32, (FT, C), 0)
        mask = (ri >= lo) & (ri < hi)
        out_ref[pl.ds(row0, FT), :] += jnp.where(mask, contrib, 0.0)


def _ffn(xs, starts, ends, w1, b1, w2, b2, g, bb):
    grid_spec = pltpu.PrefetchScalarGridSpec(
        num_scalar_prefetch=2,
        grid=(NE, NDC, NFT),
        in_specs=[
            pl.BlockSpec((T, C), lambda e, d, ti, s, en: (0, 0)),
            pl.BlockSpec((1, DC, C), lambda e, d, ti, s, en: (e, d, 0)),
            pl.BlockSpec((1, 1, DC), lambda e, d, ti, s, en: (e, 0, d)),
            pl.BlockSpec((1, C, DC), lambda e, d, ti, s, en: (e, 0, d)),
            pl.BlockSpec((1, 1, C), lambda e, d, ti, s, en: (e, 0, 0)),
            pl.BlockSpec((1, C), lambda e, d, ti, s, en: (0, 0)),
            pl.BlockSpec((1, C), lambda e, d, ti, s, en: (0, 0)),
        ],
        out_specs=pl.BlockSpec((T, C), lambda e, d, ti, s, en: (0, 0)),
        scratch_shapes=[pltpu.VMEM((T, C), jnp.float32)],
    )
    return pl.pallas_call(
        _ffn_body,
        grid_spec=grid_spec,
        out_shape=jax.ShapeDtypeStruct((T, C), jnp.float32),
    )(starts, ends, xs, w1, b1.reshape(NE, 1, DFF), w2,
      b2.reshape(NE, 1, C), g, bb)


def kernel(x, pos_table, ln1_g, ln1_b, ln2_g, ln2_b, Wq, Wk, Wv, Wo, bo,
           Wg, W1, b1, W2, b2):
    x2d = x.reshape(T, C)
    rot = _rotary_const()
    g1 = ln1_g.reshape(1, C)
    b1v = ln1_b.reshape(1, C)
    g2 = ln2_g.reshape(1, C)
    b2v = ln2_b.reshape(1, C)
    wq2 = Wq.reshape(C, C)
    wk2 = Wk.reshape(C, C)
    wv2 = Wv.reshape(C, C)
    wot = Wo.T
    bo2 = bo.reshape(1, C)
    wgp = jnp.concatenate([Wg, jnp.zeros((EPAD - NE, C), jnp.float32)], axis=0)

    q2d, k2d, v2d = _qkv(x2d, rot, pos_table, g1, b1v, wq2, wk2, wv2)
    att = _attn(q2d, k2d, v2d)
    x2, gl = _proj(x2d, att, wot, bo2, g2, b2v, wgp)
    p2d, se = _route(gl)
    p = p2d[:, 0]
    starts = se[0, :NE]
    ends = se[0, NE:2 * NE]
    xs = _sc_scatter_rows(x2, p)
    ys = _ffn(xs, starts, ends, W1, b1, W2, b2, g2, b2v)
    out = _sc_gather_rows(ys, p)
    return out.reshape(1, T, C)


# speedup vs baseline: 1.3288x; 1.3288x over previous
import functools

import numpy as np
import jax
import jax.numpy as jnp
from jax import lax
from jax.experimental import pallas as pl
from jax.experimental.pallas import tpu as pltpu
from jax.experimental.pallas import tpu_sc as plsc

T = 2048
C = 1024
H = 16
HD = 64
NE = 8
DFF = 4096
EPAD = 128

BT = 256
NT = T // BT
FT = 256
NFT = T // FT
DC = 1024
NDC = DFF // DC

SC_CORES = 2
SC_SUBCORES = 16
NW = SC_CORES * SC_SUBCORES
RPW = T // NW

_SCALE = float(C) ** -0.5
_NEG = -1e30


def _rotary_const():
    t = np.arange(T, dtype=np.float32)
    f = np.arange(0, C, 2, dtype=np.float32) / C
    ang = 2.0 * np.pi * t[:, None] * f[None, :]
    emb = np.zeros((T, C), np.float32)
    emb[:, 0::2] = np.sin(ang)
    emb[:, 1::2] = np.cos(ang)
    return jnp.asarray(emb)


def _ln(x, g, b):
    m = jnp.mean(x, axis=-1, keepdims=True)
    v = jnp.mean((x - m) ** 2, axis=-1, keepdims=True)
    return (x - m) * lax.rsqrt(v + 1e-5) * g + b


def _dot_nt(a, b):
    return lax.dot_general(a, b, (((1,), (1,)), ((), ())),
                           preferred_element_type=jnp.float32)


def _dot_nn(a, b):
    return lax.dot_general(a, b, (((1,), (0,)), ((), ())),
                           preferred_element_type=jnp.float32)


def _dot_nt_bf(a, b):
    return lax.dot_general(a.astype(jnp.bfloat16), b.astype(jnp.bfloat16),
                           (((1,), (1,)), ((), ())),
                           preferred_element_type=jnp.float32)


def _qkv_body(x_ref, rot_ref, pos_ref, g_ref, b_ref, wq_ref, wk_ref, wv_ref,
              q_ref, k_ref, v_ref):
    h = _ln(x_ref[...] + rot_ref[...] + pos_ref[...], g_ref[...], b_ref[...])
    q_ref[...] = _dot_nt(h, wq_ref[...])
    k_ref[...] = _dot_nt(h, wk_ref[...])
    v_ref[...] = _dot_nt(h, wv_ref[...])


def _qkv(x2d, rot, pos, g, b, wq2, wk2, wv2):
    row = pl.BlockSpec((BT, C), lambda i: (i, 0))
    full = pl.BlockSpec((C, C), lambda i: (0, 0))
    vec = pl.BlockSpec((1, C), lambda i: (0, 0))
    return pl.pallas_call(
        _qkv_body,
        grid=(NT,),
        in_specs=[row, row, row, vec, vec, full, full, full],
        out_specs=[row, row, row],
        out_shape=[jax.ShapeDtypeStruct((T, C), jnp.float32)] * 3,
    )(x2d, rot, pos, g, b, wq2, wk2, wv2)


HP = H // 2
AT = 512
NAT = T // AT


def _attn_body(q_ref, k_ref, v_ref, o_ref):
    tq = pl.program_id(1)
    q = q_ref[...]
    k = k_ref[...]
    v = v_ref[...]
    ri = lax.broadcasted_iota(jnp.int32, (AT, T), 0) + tq * AT
    cj = lax.broadcasted_iota(jnp.int32, (AT, T), 1)
    causal = cj <= ri
    outs = []
    for hh in range(2):
        qh = q[:, hh * HD:(hh + 1) * HD]
        kh = k[:, hh * HD:(hh + 1) * HD]
        vh = v[:, hh * HD:(hh + 1) * HD]
        s = _dot_nt(qh, kh) * _SCALE
        s = jnp.where(causal, s, _NEG)
        m = jnp.max(s, axis=1, keepdims=True)
        p = jnp.exp(s - m)
        l = jnp.sum(p, axis=1, keepdims=True)
        pv = _dot_nn(p, vh)
        outs.append(pv / l)
    o_ref[...] = jnp.concatenate(outs, axis=1)


def _attn(q2d, k2d, v2d):
    qspec = pl.BlockSpec((AT, 2 * HD), lambda hp, tq: (tq, hp))
    kvspec = pl.BlockSpec((T, 2 * HD), lambda hp, tq: (0, hp))
    return pl.pallas_call(
        _attn_body,
        grid=(HP, NAT),
        in_specs=[qspec, kvspec, kvspec],
        out_specs=pl.BlockSpec((AT, 2 * HD), lambda hp, tq: (tq, hp)),
        out_shape=jax.ShapeDtypeStruct((T, C), jnp.float32),
    )(q2d, k2d, v2d)


def _proj_body(x_ref, a_ref, wot_ref, bo_ref, g_ref, b_ref, wg_ref,
               x2_ref, gl_ref):
    x2 = x_ref[...] + _dot_nn(a_ref[...], wot_ref[...]) + bo_ref[...]
    x2_ref[...] = x2
    h2 = _ln(x2, g_ref[...], b_ref[...])
    gl_ref[...] = _dot_nt(h2, wg_ref[...])


def _proj(x2d, att, wot, bo, g, b, wgp):
    row = pl.BlockSpec((BT, C), lambda i: (i, 0))
    vec = pl.BlockSpec((1, C), lambda i: (0, 0))
    return pl.pallas_call(
        _proj_body,
        grid=(NT,),
        in_specs=[row, row,
                  pl.BlockSpec((C, C), lambda i: (0, 0)),
                  vec, vec, vec,
                  pl.BlockSpec((EPAD, C), lambda i: (0, 0))],
        out_specs=[row, pl.BlockSpec((BT, EPAD), lambda i: (i, 0))],
        out_shape=[jax.ShapeDtypeStruct((T, C), jnp.float32),
                   jax.ShapeDtypeStruct((T, EPAD), jnp.float32)],
    )(x2d, att, wot, bo, g, b, wgp)


def _route_body(gl_ref, p_ref, se_ref):
    gl = gl_ref[...]
    cols = lax.broadcasted_iota(jnp.int32, (T, EPAD), 1)
    gl = jnp.where(cols < NE, gl, _NEG)
    m = jnp.max(gl, axis=1, keepdims=True)
    sel = jnp.min(jnp.where(gl == m, cols, EPAD), axis=1, keepdims=True)
    oh = (cols == sel).astype(jnp.float32)
    ri = lax.broadcasted_iota(jnp.int32, (T, T), 0)
    ci = lax.broadcasted_iota(jnp.int32, (T, T), 1)
    lmat = (ci < ri).astype(jnp.float32)
    cs = lax.dot_general(lmat, oh, (((1,), (0,)), ((), ())),
                         preferred_element_type=jnp.float32,
                         precision=lax.Precision.HIGHEST)
    rank = jnp.sum(oh * cs, axis=1, keepdims=True)
    counts = jnp.sum(oh, axis=0, keepdims=True)
    ei = lax.broadcasted_iota(jnp.int32, (EPAD, EPAD), 0)
    ej = lax.broadcasted_iota(jnp.int32, (EPAD, EPAD), 1)
    umat = (ei < ej).astype(jnp.float32)
    offs = lax.dot_general(counts, umat, (((1,), (0,)), ((), ())),
                           preferred_element_type=jnp.float32,
                           precision=lax.Precision.HIGHEST)
    pos = jnp.sum(oh * offs, axis=1, keepdims=True) + rank
    p_ref[...] = jnp.broadcast_to(pos.astype(jnp.int32), (T, EPAD))
    ends = offs + counts
    shmat = ((ei + NE) == ej).astype(jnp.float32)
    ends_sh = lax.dot_general(ends, shmat, (((1,), (0,)), ((), ())),
                              preferred_element_type=jnp.float32,
                              precision=lax.Precision.HIGHEST)
    ecol = lax.broadcasted_iota(jnp.int32, (1, EPAD), 1)
    packed = jnp.where(ecol < NE, offs,
                       jnp.where(ecol < 2 * NE, ends_sh, 0.0))
    se_ref[...] = packed.astype(jnp.int32)


def _route(gl):
    return pl.pallas_call(
        _route_body,
        grid=(1,),
        in_specs=[pl.BlockSpec((T, EPAD), lambda i: (0, 0))],
        out_specs=[pl.BlockSpec((T, EPAD), lambda i: (0, 0)),
                   pl.BlockSpec((1, EPAD), lambda i: (0, 0))],
        out_shape=[jax.ShapeDtypeStruct((T, EPAD), jnp.int32),
                   jax.ShapeDtypeStruct((1, EPAD), jnp.int32)],
    )(gl)


@functools.lru_cache(maxsize=None)
def _sc_kernels():
    mesh = plsc.VectorSubcoreMesh(core_axis_name="c", subcore_axis_name="s")
    common = dict(
        out_type=jax.ShapeDtypeStruct((T, C), jnp.float32),
        mesh=mesh,
        scratch_types=[
            pltpu.VMEM((RPW,), jnp.int32),
            pltpu.VMEM((RPW, C), jnp.float32),
            pltpu.SemaphoreType.DMA,
        ],
    )

    @functools.partial(pl.kernel, **common)
    def scatter_rows(x_hbm, p_hbm, out_hbm, idx_v, rows_v, sem):
        wid = lax.axis_index("s") * SC_CORES + lax.axis_index("c")
        base = wid * RPW
        pltpu.sync_copy(p_hbm.at[pl.ds(base, RPW)], idx_v)
        pltpu.sync_copy(x_hbm.at[pl.ds(base, RPW)], rows_v)
        pltpu.async_copy(rows_v, out_hbm.at[idx_v], sem).wait()

    @functools.partial(pl.kernel, **common)
    def gather_rows(ys_hbm, p_hbm, out_hbm, idx_v, rows_v, sem):
        wid = lax.axis_index("s") * SC_CORES + lax.axis_index("c")
        base = wid * RPW
        pltpu.sync_copy(p_hbm.at[pl.ds(base, RPW)], idx_v)
        pltpu.async_copy(ys_hbm.at[idx_v], rows_v, sem).wait()
        pltpu.sync_copy(rows_v, out_hbm.at[pl.ds(base, RPW)])

    return scatter_rows, gather_rows


def _sc_scatter_rows(x2, p):
    return _sc_kernels()[0](x2, p)


def _sc_gather_rows(ys, p):
    return _sc_kernels()[1](ys, p)


def _ffn_body(s_ref, e_ref, xs_ref, w1_ref, b1_ref, w2_ref, b2_ref,
              g_ref, b_ref, out_ref, h2_ref):
    e = pl.program_id(0)
    d = pl.program_id(1)
    ti = pl.program_id(2)

    @pl.when((e == 0) & (d == 0) & (ti == 0))
    def _init():
        xs = xs_ref[...]
        out_ref[...] = xs
        h2_ref[...] = _ln(xs, g_ref[...], b_ref[...])

    seg_s = s_ref[e]
    seg_e = e_ref[e]
    t = jnp.minimum(seg_s // FT + ti, NFT - 1)
    row0 = t * FT
    valid = (seg_s < seg_e) & (seg_s // FT + ti <= (seg_e - 1) // FT)

    @pl.when(valid)
    def _step():
        h2 = h2_ref[pl.ds(row0, FT), :]
        a = jnp.maximum(_dot_nt_bf(h2, w1_ref[0]) + b1_ref[0], 0.0)
        contrib = _dot_nt_bf(a, w2_ref[0])
        contrib = contrib + jnp.where(d == 0, 1.0, 0.0) * b2_ref[0]
        lo = jnp.maximum(seg_s - row0, 0)
        hi = jnp.minimum(seg_e - row0, FT)
        ri = lax.broadcasted_iota(jnp.int32, (FT, C), 0)
        mask = (ri >= lo) & (ri < hi)
        out_ref[pl.ds(row0, FT), :] += jnp.where(mask, contrib, 0.0)


def _ffn(xs, starts, ends, w1, b1, w2, b2, g, bb):
    grid_spec = pltpu.PrefetchScalarGridSpec(
        num_scalar_prefetch=2,
        grid=(NE, NDC, NFT),
        in_specs=[
            pl.BlockSpec((T, C), lambda e, d, ti, s, en: (0, 0)),
            pl.BlockSpec((1, DC, C), lambda e, d, ti, s, en: (e, d, 0)),
            pl.BlockSpec((1, 1, DC), lambda e, d, ti, s, en: (e, 0, d)),
            pl.BlockSpec((1, C, DC), lambda e, d, ti, s, en: (e, 0, d)),
            pl.BlockSpec((1, 1, C), lambda e, d, ti, s, en: (e, 0, 0)),
            pl.BlockSpec((1, C), lambda e, d, ti, s, en: (0, 0)),
            pl.BlockSpec((1, C), lambda e, d, ti, s, en: (0, 0)),
        ],
        out_specs=pl.BlockSpec((T, C), lambda e, d, ti, s, en: (0, 0)),
        scratch_shapes=[pltpu.VMEM((T, C), jnp.float32)],
    )
    return pl.pallas_call(
        _ffn_body,
        grid_spec=grid_spec,
        out_shape=jax.ShapeDtypeStruct((T, C), jnp.float32),
    )(starts, ends, xs, w1, b1.reshape(NE, 1, DFF), w2,
      b2.reshape(NE, 1, C), g, bb)


def kernel(x, pos_table, ln1_g, ln1_b, ln2_g, ln2_b, Wq, Wk, Wv, Wo, bo,
           Wg, W1, b1, W2, b2):
    x2d = x.reshape(T, C)
    rot = _rotary_const()
    g1 = ln1_g.reshape(1, C)
    b1v = ln1_b.reshape(1, C)
    g2 = ln2_g.reshape(1, C)
    b2v = ln2_b.reshape(1, C)
    wq2 = Wq.reshape(C, C)
    wk2 = Wk.reshape(C, C)
    wv2 = Wv.reshape(C, C)
    wot = Wo.T
    bo2 = bo.reshape(1, C)
    wgp = jnp.concatenate([Wg, jnp.zeros((EPAD - NE, C), jnp.float32)], axis=0)

    q2d, k2d, v2d = _qkv(x2d, rot, pos_table, g1, b1v, wq2, wk2, wv2)
    att = _attn(q2d, k2d, v2d)
    x2, gl = _proj(x2d, att, wot, bo2, g2, b2v, wgp)
    p2d, se = _route(gl)
    p = p2d[:, 0]
    starts = se[0, :NE]
    ends = se[0, NE:2 * NE]
    xs = _sc_scatter_rows(x2, p)
    ys = _ffn(xs, starts, ends, W1, b1, W2, b2, g2, b2v)
    out = _sc_gather_rows(ys, p)
    return out.reshape(1, T, C)
